# Initial kernel scaffold; baseline (speedup 1.0000x reference)
import jax
import jax.numpy as jnp
from jax.experimental import pallas as pl


def kernel(X, inds, vals):
    raise NotImplementedError("write your pallas kernel here")


# speedup vs baseline: 5.1586x; 5.1586x over previous
import functools

import jax
import jax.numpy as jnp
from jax import lax
from jax.experimental import pallas as pl
from jax.experimental.pallas import tpu as pltpu
from jax.experimental.pallas import tpu_sc as plsc

ORIG_SHAPE = (4096, 4096)
NUMEL = ORIG_SHAPE[0] * ORIG_SHAPE[1]
K = 1677721

_info = plsc.get_sparse_core_info()
NC = _info.num_cores
NS = _info.num_subcores
NW = NC * NS

PER_W = 52480
K_PAD = PER_W * NW
PAD = K_PAD - K

ROWS_PER_BLOCK = 256
N_BLOCKS = ORIG_SHAPE[0] // ROWS_PER_BLOCK


def _copy_body(x_ref, o_ref):
    o_ref[...] = x_ref[...]


_copy = pl.pallas_call(
    _copy_body,
    grid=(N_BLOCKS,),
    in_specs=[pl.BlockSpec((ROWS_PER_BLOCK, ORIG_SHAPE[1]), lambda i: (i, 0))],
    out_specs=pl.BlockSpec((ROWS_PER_BLOCK, ORIG_SHAPE[1]), lambda i: (i, 0)),
    out_shape=jax.ShapeDtypeStruct(ORIG_SHAPE, jnp.float32),
)

_mesh = plsc.VectorSubcoreMesh(core_axis_name="c", subcore_axis_name="s")


@functools.partial(
    pl.kernel,
    mesh=_mesh,
    out_type=(),
    scratch_types=[
        pltpu.VMEM((PER_W,), jnp.int32),
        pltpu.VMEM((PER_W,), jnp.float32),
        pltpu.SemaphoreType.DMA,
    ],
)
def _scatter(y_hbm, inds_hbm, vals_hbm, idx_v, val_v, sem):
    wid = lax.axis_index("s") * NC + lax.axis_index("c")
    base = wid * PER_W
    pltpu.sync_copy(inds_hbm.at[pl.ds(base, PER_W)], idx_v)
    pltpu.sync_copy(vals_hbm.at[pl.ds(base, PER_W)], val_v)
    pltpu.async_copy(val_v, y_hbm.at[idx_v], sem).wait()


def kernel(X, inds, vals):
    y = _copy(X).reshape(-1)
    inds_p = jnp.concatenate([inds, inds[:PAD]])
    vals_p = jnp.concatenate([vals, vals[:PAD]])
    y_ref = jax.new_ref(y)
    _scatter(y_ref, inds_p, vals_p)
    return y_ref[...].reshape(ORIG_SHAPE)
